# baseline (speedup 1.0000x reference)
import jax
import jax.numpy as jnp
from jax.experimental import pallas as pl


def kernel(x, pos_emb):
    batch = x.shape[0]
    max_len, d_model = pos_emb.shape
    block_rows = 1024

    def body(p_ref, o_ref):
        o_ref[...] = p_ref[...][None, :, :]

    return pl.pallas_call(
        body,
        grid=(max_len // block_rows, batch),
        in_specs=[pl.BlockSpec((block_rows, d_model), lambda i, b: (i, 0))],
        out_specs=pl.BlockSpec(
            (1, block_rows, d_model), lambda i, b: (b, i, 0)
        ),
        out_shape=jax.ShapeDtypeStruct((batch, max_len, d_model), pos_emb.dtype),
    )(pos_emb)


# speedup vs baseline: 1.2331x; 1.2331x over previous
import jax
import jax.numpy as jnp
from jax.experimental import pallas as pl


def kernel(x, pos_emb):
    batch = x.shape[0]
    max_len, d_model = pos_emb.shape
    block_rows = 1024

    def body(p_ref, o_ref):
        blk = p_ref[...]
        o_ref[...] = jnp.broadcast_to(blk[None, :, :], (batch, block_rows, d_model))

    return pl.pallas_call(
        body,
        grid=(max_len // block_rows,),
        in_specs=[pl.BlockSpec((block_rows, d_model), lambda i: (i, 0))],
        out_specs=pl.BlockSpec(
            (batch, block_rows, d_model), lambda i: (0, i, 0)
        ),
        out_shape=jax.ShapeDtypeStruct((batch, max_len, d_model), pos_emb.dtype),
    )(pos_emb)
